# Initial kernel scaffold; baseline (speedup 1.0000x reference)
import jax
import jax.numpy as jnp
from jax.experimental import pallas as pl


def kernel(f_in, pos, batch, node_atom, emb, Wf, bf, Wp1, bp1, Wp2, bp2, U, V, Wu1, bu1, Wu2, bu2, Wo1, bo1, Wo2, bo2):
    raise NotImplementedError("write your pallas kernel here")


# speedup vs baseline: 27.3865x; 27.3865x over previous
import jax
import jax.numpy as jnp
from jax import lax
from jax.experimental import pallas as pl
from jax.experimental.pallas import tpu as pltpu

H = 128
RBF = 64
CUT = 5.0
BMAX = 64
GAMMA = 1.0 / ((CUT / RBF) ** 2 + 1e-09)

BR = 32
BC = 128
BN = 512


def _silu(x):
    return x * jax.nn.sigmoid(x)


def _init_body(na_ref, emb_ref, wp1_ref, bp1_ref, wp2_ref, bp2_ref,
               s_ref, phi_ref):
    na = na_ref[...]
    zmax = emb_ref.shape[0]
    oh = (na == lax.broadcasted_iota(jnp.int32, (na.shape[0], zmax), 1))
    s = jnp.dot(oh.astype(jnp.float32), emb_ref[...],
                preferred_element_type=jnp.float32)
    t = _silu(jnp.dot(s, wp1_ref[...], preferred_element_type=jnp.float32)
              + bp1_ref[...])
    phi_ref[...] = jnp.dot(t, wp2_ref[...], preferred_element_type=jnp.float32) \
        + bp2_ref[...]
    s_ref[...] = s


def _msg_body(lo_ref, nb_ref,
              posr_ref, batr_ref, s_ref, v0r_ref, v1r_ref, v2r_ref,
              post_ref, batt_ref, phi_ref, v0_ref, v1_ref, v2_ref,
              wf_ref, bf_ref,
              so_ref, v0o_ref, v1o_ref, v2o_ref):
    i = pl.program_id(0)
    r0 = i * BR
    pr = posr_ref[...]
    prx = pr[:, 0:1]
    pry = pr[:, 1:2]
    prz = pr[:, 2:3]
    br = batr_ref[...]
    lo = lo_ref[i]
    nb = nb_ref[i]
    centers3 = lax.broadcasted_iota(jnp.int32, (1, 1, RBF), 2).astype(
        jnp.float32) * (CUT / (RBF - 1))
    riota = lax.broadcasted_iota(jnp.int32, (BR, BC), 0) + r0
    ciota = lax.broadcasted_iota(jnp.int32, (BR, BC), 1)
    bfs3 = bf_ref[:, 0:H].reshape(1, 1, H)
    bfvv3 = bf_ref[:, H:2 * H].reshape(1, 1, H)
    bfvs3 = bf_ref[:, 2 * H:3 * H].reshape(1, 1, H)

    def body(t, carry):
        ds, dv0, dv1, dv2 = carry
        c0 = (lo + t) * BC
        pcx = post_ref[0:1, pl.ds(c0, BC)]
        pcy = post_ref[1:2, pl.ds(c0, BC)]
        pcz = post_ref[2:3, pl.ds(c0, BC)]
        dx = prx - pcx
        dy = pry - pcy
        dz = prz - pcz
        dist = jnp.sqrt(dx * dx + dy * dy + dz * dz + 1e-12)
        inv = 1.0 / dist
        fcut = 0.5 * (jnp.cos(jnp.pi / CUT * dist) + 1.0) \
            * (dist < CUT).astype(jnp.float32)
        bc = batt_ref[0:1, pl.ds(c0, BC)]
        mask = (br == bc) & (riota != ciota + c0)
        wm = fcut * mask.astype(jnp.float32)
        wm3 = wm[:, :, None]
        ea3 = jnp.exp(-GAMMA * (dist[:, :, None] - centers3) ** 2) * wm3
        ea = ea3.reshape(BR * BC, RBF)
        phis = phi_ref[pl.ds(c0, BC), 0:H]
        phvv = phi_ref[pl.ds(c0, BC), H:2 * H]
        phvs = phi_ref[pl.ds(c0, BC), 2 * H:3 * H]
        vc0 = v0_ref[pl.ds(c0, BC), :]
        vc1 = v1_ref[pl.ds(c0, BC), :]
        vc2 = v2_ref[pl.ds(c0, BC), :]
        ws3 = jnp.dot(ea, wf_ref[:, 0:H],
                      preferred_element_type=jnp.float32).reshape(BR, BC, H) \
            + bfs3 * wm3
        ds = ds + jnp.sum(ws3 * phis[None, :, :], axis=1)
        wvv3 = jnp.dot(ea, wf_ref[:, H:2 * H],
                       preferred_element_type=jnp.float32).reshape(BR, BC, H) \
            + bfvv3 * wm3
        dv0 = dv0 + jnp.sum(wvv3 * (phvv * vc0)[None, :, :], axis=1)
        dv1 = dv1 + jnp.sum(wvv3 * (phvv * vc1)[None, :, :], axis=1)
        dv2 = dv2 + jnp.sum(wvv3 * (phvv * vc2)[None, :, :], axis=1)
        wvs3 = jnp.dot(ea, wf_ref[:, 2 * H:3 * H],
                       preferred_element_type=jnp.float32).reshape(BR, BC, H) \
            + bfvs3 * wm3
        fvs = wvs3 * phvs[None, :, :]
        dv0 = dv0 + jnp.sum((dx * inv)[:, :, None] * fvs, axis=1)
        dv1 = dv1 + jnp.sum((dy * inv)[:, :, None] * fvs, axis=1)
        dv2 = dv2 + jnp.sum((dz * inv)[:, :, None] * fvs, axis=1)
        return ds, dv0, dv1, dv2

    z = jnp.zeros((BR, H), jnp.float32)
    ds, dv0, dv1, dv2 = lax.fori_loop(0, nb, body, (z, z, z, z))
    so_ref[...] = s_ref[...] + ds
    v0o_ref[...] = v0r_ref[...] + dv0
    v1o_ref[...] = v1r_ref[...] + dv1
    v2o_ref[...] = v2r_ref[...] + dv2


def _upd_body(s_ref, v0_ref, v1_ref, v2_ref,
              u_ref, vm_ref, wu1a_ref, wu1b_ref, bu1_ref, wu2_ref, bu2_ref,
              wp1_ref, bp1_ref, wp2_ref, bp2_ref,
              so_ref, v0o_ref, v1o_ref, v2o_ref, phio_ref):
    s = s_ref[...]
    v0 = v0_ref[...]
    v1 = v1_ref[...]
    v2 = v2_ref[...]
    U = u_ref[...]
    Vm = vm_ref[...]
    vu0 = jnp.dot(v0, U, preferred_element_type=jnp.float32)
    vu1 = jnp.dot(v1, U, preferred_element_type=jnp.float32)
    vu2 = jnp.dot(v2, U, preferred_element_type=jnp.float32)
    vv0 = jnp.dot(v0, Vm, preferred_element_type=jnp.float32)
    vv1 = jnp.dot(v1, Vm, preferred_element_type=jnp.float32)
    vv2 = jnp.dot(v2, Vm, preferred_element_type=jnp.float32)
    vvn = jnp.sqrt(vv0 * vv0 + vv1 * vv1 + vv2 * vv2 + 1e-08)
    pre = jnp.dot(s, wu1a_ref[...], preferred_element_type=jnp.float32) \
        + jnp.dot(vvn, wu1b_ref[...], preferred_element_type=jnp.float32) \
        + bu1_ref[...]
    a = jnp.dot(_silu(pre), wu2_ref[...], preferred_element_type=jnp.float32) \
        + bu2_ref[...]
    ass = a[:, 0:H]
    asv = a[:, H:2 * H]
    avv = a[:, 2 * H:3 * H]
    sn = s + ass + asv * (vu0 * vv0 + vu1 * vv1 + vu2 * vv2)
    so_ref[...] = sn
    v0o_ref[...] = v0 + vu0 * avv
    v1o_ref[...] = v1 + vu1 * avv
    v2o_ref[...] = v2 + vu2 * avv
    t = _silu(jnp.dot(sn, wp1_ref[...], preferred_element_type=jnp.float32)
              + bp1_ref[...])
    phio_ref[...] = jnp.dot(t, wp2_ref[...], preferred_element_type=jnp.float32) \
        + bp2_ref[...]


def _final_body(s_ref, bat_ref, wo1_ref, bo1_ref, wo2t_ref, bo2_ref, out_ref):
    i = pl.program_id(0)

    @pl.when(i == 0)
    def _():
        out_ref[...] = jnp.zeros_like(out_ref)

    t = _silu(jnp.dot(s_ref[...], wo1_ref[...],
                      preferred_element_type=jnp.float32) + bo1_ref[...])
    pa = jnp.sum(t * wo2t_ref[...], axis=1, keepdims=True) + bo2_ref[...]
    oh = (bat_ref[...] == lax.broadcasted_iota(jnp.int32, (BN, BMAX), 1))
    out_ref[...] = out_ref[...] + jnp.sum(oh.astype(jnp.float32) * pa, axis=0,
                                          keepdims=True)


def kernel(f_in, pos, batch, node_atom, emb, Wf, bf, Wp1, bp1, Wp2, bp2,
           U, V, Wu1, bu1, Wu2, bu2, Wo1, bo1, Wo2, bo2):
    N = pos.shape[0]
    L = Wf.shape[0]
    NR = N // BR
    NBN = N // BN

    batch = batch.astype(jnp.int32)
    node_atom = node_atom.astype(jnp.int32)
    bat2d = batch.reshape(N, 1)
    batT = batch.reshape(1, N)
    posT = pos.T

    b_first = batch[::BR]
    b_last = batch[BR - 1::BR]
    lo = jnp.searchsorted(batch, b_first, side='left').astype(jnp.int32)
    hi = jnp.searchsorted(batch, b_last, side='right').astype(jnp.int32)
    lo_blk = lo // BC
    n_blk = (hi + BC - 1) // BC - lo_blk

    na2d = node_atom.reshape(N, 1)
    s, phi = pl.pallas_call(
        _init_body,
        grid=(NBN,),
        in_specs=[
            pl.BlockSpec((BN, 1), lambda i: (i, 0)),
            pl.BlockSpec(emb.shape, lambda i: (0, 0)),
            pl.BlockSpec((H, H), lambda i: (0, 0)),
            pl.BlockSpec((1, H), lambda i: (0, 0)),
            pl.BlockSpec((H, 3 * H), lambda i: (0, 0)),
            pl.BlockSpec((1, 3 * H), lambda i: (0, 0)),
        ],
        out_specs=[
            pl.BlockSpec((BN, H), lambda i: (i, 0)),
            pl.BlockSpec((BN, 3 * H), lambda i: (i, 0)),
        ],
        out_shape=[
            jax.ShapeDtypeStruct((N, H), jnp.float32),
            jax.ShapeDtypeStruct((N, 3 * H), jnp.float32),
        ],
        compiler_params=pltpu.CompilerParams(
            dimension_semantics=("parallel",)),
    )(na2d, emb, Wp1[0], bp1[0].reshape(1, H), Wp2[0], bp2[0].reshape(1, 3 * H))

    v0 = jnp.zeros((N, H), jnp.float32)
    v1 = jnp.zeros((N, H), jnp.float32)
    v2 = jnp.zeros((N, H), jnp.float32)

    msg_grid = pltpu.PrefetchScalarGridSpec(
        num_scalar_prefetch=2,
        grid=(NR,),
        in_specs=[
            pl.BlockSpec((BR, 3), lambda i, lo, nb: (i, 0)),
            pl.BlockSpec((BR, 1), lambda i, lo, nb: (i, 0)),
            pl.BlockSpec((BR, H), lambda i, lo, nb: (i, 0)),
            pl.BlockSpec((BR, H), lambda i, lo, nb: (i, 0)),
            pl.BlockSpec((BR, H), lambda i, lo, nb: (i, 0)),
            pl.BlockSpec((BR, H), lambda i, lo, nb: (i, 0)),
            pl.BlockSpec((3, N), lambda i, lo, nb: (0, 0)),
            pl.BlockSpec((1, N), lambda i, lo, nb: (0, 0)),
            pl.BlockSpec((N, 3 * H), lambda i, lo, nb: (0, 0)),
            pl.BlockSpec((N, H), lambda i, lo, nb: (0, 0)),
            pl.BlockSpec((N, H), lambda i, lo, nb: (0, 0)),
            pl.BlockSpec((N, H), lambda i, lo, nb: (0, 0)),
            pl.BlockSpec((RBF, 3 * H), lambda i, lo, nb: (0, 0)),
            pl.BlockSpec((1, 3 * H), lambda i, lo, nb: (0, 0)),
        ],
        out_specs=[
            pl.BlockSpec((BR, H), lambda i, lo, nb: (i, 0)),
            pl.BlockSpec((BR, H), lambda i, lo, nb: (i, 0)),
            pl.BlockSpec((BR, H), lambda i, lo, nb: (i, 0)),
            pl.BlockSpec((BR, H), lambda i, lo, nb: (i, 0)),
        ],
    )
    msg_call = pl.pallas_call(
        _msg_body,
        grid_spec=msg_grid,
        out_shape=[jax.ShapeDtypeStruct((N, H), jnp.float32)] * 4,
        compiler_params=pltpu.CompilerParams(
            dimension_semantics=("arbitrary",)),
    )

    upd_call = pl.pallas_call(
        _upd_body,
        grid=(NBN,),
        in_specs=[
            pl.BlockSpec((BN, H), lambda i: (i, 0)),
            pl.BlockSpec((BN, H), lambda i: (i, 0)),
            pl.BlockSpec((BN, H), lambda i: (i, 0)),
            pl.BlockSpec((BN, H), lambda i: (i, 0)),
            pl.BlockSpec((H, H), lambda i: (0, 0)),
            pl.BlockSpec((H, H), lambda i: (0, 0)),
            pl.BlockSpec((H, H), lambda i: (0, 0)),
            pl.BlockSpec((H, H), lambda i: (0, 0)),
            pl.BlockSpec((1, H), lambda i: (0, 0)),
            pl.BlockSpec((H, 3 * H), lambda i: (0, 0)),
            pl.BlockSpec((1, 3 * H), lambda i: (0, 0)),
            pl.BlockSpec((H, H), lambda i: (0, 0)),
            pl.BlockSpec((1, H), lambda i: (0, 0)),
            pl.BlockSpec((H, 3 * H), lambda i: (0, 0)),
            pl.BlockSpec((1, 3 * H), lambda i: (0, 0)),
        ],
        out_specs=[
            pl.BlockSpec((BN, H), lambda i: (i, 0)),
            pl.BlockSpec((BN, H), lambda i: (i, 0)),
            pl.BlockSpec((BN, H), lambda i: (i, 0)),
            pl.BlockSpec((BN, H), lambda i: (i, 0)),
            pl.BlockSpec((BN, 3 * H), lambda i: (i, 0)),
        ],
        out_shape=[jax.ShapeDtypeStruct((N, H), jnp.float32)] * 4
        + [jax.ShapeDtypeStruct((N, 3 * H), jnp.float32)],
        compiler_params=pltpu.CompilerParams(
            dimension_semantics=("parallel",)),
    )

    for l in range(L):
        s, v0, v1, v2 = msg_call(
            lo_blk, n_blk,
            pos, bat2d, s, v0, v1, v2,
            posT, batT, phi, v0, v1, v2,
            Wf[l], bf[l].reshape(1, 3 * H))
        ln = (l + 1) % L
        s, v0, v1, v2, phi = upd_call(
            s, v0, v1, v2,
            U[l], V[l], Wu1[l][:H], Wu1[l][H:], bu1[l].reshape(1, H),
            Wu2[l], bu2[l].reshape(1, 3 * H),
            Wp1[ln], bp1[ln].reshape(1, H), Wp2[ln], bp2[ln].reshape(1, 3 * H))

    out = pl.pallas_call(
        _final_body,
        grid=(NBN,),
        in_specs=[
            pl.BlockSpec((BN, H), lambda i: (i, 0)),
            pl.BlockSpec((BN, 1), lambda i: (i, 0)),
            pl.BlockSpec((H, H // 2), lambda i: (0, 0)),
            pl.BlockSpec((1, H // 2), lambda i: (0, 0)),
            pl.BlockSpec((1, H // 2), lambda i: (0, 0)),
            pl.BlockSpec((1, 1), lambda i: (0, 0)),
        ],
        out_specs=pl.BlockSpec((1, BMAX), lambda i: (0, 0)),
        out_shape=jax.ShapeDtypeStruct((1, BMAX), jnp.float32),
        compiler_params=pltpu.CompilerParams(
            dimension_semantics=("arbitrary",)),
    )(s, bat2d, Wo1, bo1.reshape(1, H // 2), Wo2.reshape(1, H // 2),
      bo2.reshape(1, 1))
    return out.reshape(BMAX, 1)
